# baseline (speedup 1.0000x reference)
import functools

import jax
import jax.numpy as jnp
from jax.experimental import pallas as pl
from jax.experimental.pallas import tpu as pltpu

_K = 8
_NEG = -1e30


def _fused_kernel(x_ref, gain_ref, bias_ref, lm_ref, ls_ref, out_ref, xn_ref,
                  w_ref, msg_ref, *, blk):
    i = pl.program_id(1)

    @pl.when(i == 0)
    def _normalize():
        xa_full = x_ref[0]
        n2 = jnp.sum(xa_full * xa_full, axis=1, keepdims=True)
        xn_ref[...] = xa_full / (jnp.sqrt(n2) + 1e-8)

    row0 = i * blk
    xn_rows = xn_ref[pl.ds(row0, blk), :]

    def _col(c):
        return pl.multiple_of(c * blk, blk)

    def _sim_chunk(c):
        xc = xn_ref[pl.ds(_col(c), blk), :]
        return jax.lax.dot_general(
            xn_rows, xc, (((1,), (1,)), ((), ())),
            preferred_element_type=jnp.float32)

    def _build(c, carry):
        w_ref[:, pl.ds(_col(c), blk)] = _sim_chunk(c)
        return carry

    jax.lax.fori_loop(0, i, _build, 0, unroll=False)

    cols = jax.lax.broadcasted_iota(jnp.int32, (blk, blk), 1)
    rows = jax.lax.broadcasted_iota(jnp.int32, (blk, blk), 0)
    w_ref[:, pl.ds(_col(i), blk)] = jnp.where(cols <= rows, _sim_chunk(i), _NEG)

    nchunks = i + 1

    def _masked_max(m):
        def body(c, acc):
            wc = w_ref[:, pl.ds(_col(c), blk)]
            t = wc if m is None else jnp.where(wc < m, wc, _NEG)
            return jnp.maximum(acc, jnp.max(t, axis=1, keepdims=True))
        return jax.lax.fori_loop(0, nchunks, body,
                                 jnp.full((blk, 1), _NEG, jnp.float32))

    m = _masked_max(None)
    m = jax.lax.fori_loop(0, _K - 1, lambda r, mm: _masked_max(mm), m)
    thresh = jnp.maximum(m, -2.0)

    msg_ref[...] = jnp.zeros_like(msg_ref)

    def _agg(c, deg_acc):
        wc = w_ref[:, pl.ds(_col(c), blk)]
        adjc = jnp.where(wc >= thresh, 1.0, 0.0)
        xc = x_ref[0, pl.ds(_col(c), blk), :]
        msg_ref[...] += jax.lax.dot_general(
            adjc, xc, (((1,), (0,)), ((), ())),
            preferred_element_type=jnp.float32)
        return deg_acc + jnp.sum(adjc, axis=1, keepdims=True)

    deg = jax.lax.fori_loop(0, nchunks, _agg,
                            jnp.zeros((blk, 1), jnp.float32))

    msg = msg_ref[...] / jnp.maximum(deg, 1.0)

    mix = jax.nn.sigmoid(lm_ref[0, 0])
    scale = jax.nn.softplus(ls_ref[0, 0]) + 0.01

    x_rows = x_ref[0, pl.ds(row0, blk), :]
    blended = mix * x_rows + (1.0 - mix) * msg
    y = blended * gain_ref[0][None, :] + bias_ref[0][None, :]
    gelu = 0.5 * y * (1.0 + jax.lax.erf(y * (2.0 ** -0.5)))
    out_ref[0] = gelu * scale


def kernel(x, gain, bias, log_mix, log_scale):
    B, T, D = x.shape
    blk = min(256, T)
    grid = (B, T // blk)

    fn = functools.partial(_fused_kernel, blk=blk)
    return pl.pallas_call(
        fn,
        grid=grid,
        in_specs=[
            pl.BlockSpec((1, T, D), lambda b, i: (b, 0, 0)),
            pl.BlockSpec((1, D), lambda b, i: (0, 0)),
            pl.BlockSpec((1, D), lambda b, i: (0, 0)),
            pl.BlockSpec((1, 1), lambda b, i: (0, 0)),
            pl.BlockSpec((1, 1), lambda b, i: (0, 0)),
        ],
        out_specs=pl.BlockSpec((1, blk, D), lambda b, i: (b, i, 0)),
        out_shape=jax.ShapeDtypeStruct((B, T, D), x.dtype),
        scratch_shapes=[
            pltpu.VMEM((T, D), jnp.float32),
            pltpu.VMEM((blk, T), jnp.float32),
            pltpu.VMEM((blk, D), jnp.float32),
        ],
    )(x, gain.reshape(1, D), bias.reshape(1, D),
      log_mix.reshape(1, 1), log_scale.reshape(1, 1))
